# Initial kernel scaffold; baseline (speedup 1.0000x reference)
import jax
import jax.numpy as jnp
from jax.experimental import pallas as pl


def kernel(x, edge_index, batch, W1, b1, W2, b2, Wfc, bfc):
    raise NotImplementedError("write your pallas kernel here")


# speedup vs baseline: 1.4959x; 1.4959x over previous
import functools

import jax
import jax.numpy as jnp
from jax import lax
from jax.experimental import pallas as pl
from jax.experimental.pallas import tpu as pltpu
from jax.experimental.pallas import tpu_sc as plsc

N = 10000
NPAD = 10240
E = 160000
D = 256
HD = 128
NG = 64
NCLS = 10
NCORES = 2
NSUB = 16
LANES = 16
NQUART = 4
NQ = NPAD // NQUART
ACCR = NQ + 8
SSTR = NQ // NSUB
K = 128
CH_A = 80
EPT = CH_A * K
EPAD = NSUB * EPT
NBUF = 4
SRC_PAD = N
DST_PAD = NPAD
DGR = NPAD // 8
DACCR = DGR + 8
DSTR = DGR // NSUB
CH_D = CH_A // NCORES

_mesh = plsc.VectorSubcoreMesh(core_axis_name="c", subcore_axis_name="s")


@functools.partial(
    pl.kernel,
    out_type=jax.ShapeDtypeStruct((NCORES * DGR, HD), jnp.float32),
    mesh=_mesh,
    scratch_types=[
        pltpu.VMEM((CH_A, K), jnp.int32),
        pltpu.VMEM((CH_D, K), jnp.int32),
        pltpu.VMEM((CH_D, K), jnp.int32),
        pltpu.VMEM((NBUF, K, HD), jnp.float32),
        pltpu.VMEM_SHARED((DACCR, HD), jnp.float32),
        pltpu.SemaphoreType.DMA,
        pltpu.SemaphoreType.DMA,
        pltpu.SemaphoreType.DMA,
        pltpu.SemaphoreType.DMA,
    ],
)
def _deg_kernel(pat_hbm, dst_hbm, zeros_hbm, out_hbm,
                dst_raw, gsel, gdst, bufs, acc, s0, s1, s2, s3):
    cid = lax.axis_index("c")
    sid = lax.axis_index("s")
    sems = (s0, s1, s2, s3)
    nsl = K // 16

    pltpu.sync_copy(dst_hbm.at[sid], dst_raw)

    def _fix(t, _):
        j = t // nsl
        kk = (t % nsl) * 16
        v = dst_raw[cid * CH_D + j, pl.ds(kk, 16)]
        gsel[j, pl.ds(kk, 16)] = jnp.bitwise_and(v, 7)
        gdst[j, pl.ds(kk, 16)] = lax.shift_right_logical(v, 3)
        return _

    lax.fori_loop(0, CH_D * nsl, _fix, None)

    pltpu.sync_copy(
        zeros_hbm.at[pl.ds(sid * DSTR, DSTR)],
        acc.at[pl.ds(sid * DSTR, DSTR)],
    )

    @pl.when(sid == 0)
    def _zero_pad():
        pltpu.sync_copy(zeros_hbm.at[pl.ds(0, 8)], acc.at[pl.ds(DGR, 8)])

    plsc.subcore_barrier()

    def _group(g, _):
        descs = []
        for b in range(NBUF):
            j = NBUF * g + b
            descs.append(
                pltpu.async_copy(pat_hbm.at[gsel.at[j]], bufs.at[b], sems[b])
            )
        for b in range(NBUF):
            descs[b].wait()
        for b in range(NBUF):
            j = NBUF * g + b
            pltpu.sync_copy(bufs.at[b], acc.at[gdst.at[j]], add=True)
        return _

    lax.fori_loop(0, CH_D // NBUF, _group, None)
    plsc.subcore_barrier()
    pltpu.sync_copy(
        acc.at[pl.ds(sid * DSTR, DSTR)],
        out_hbm.at[pl.ds(cid * DGR + sid * DSTR, DSTR)],
    )


@functools.partial(
    pl.kernel,
    out_type=jax.ShapeDtypeStruct((NCORES * NPAD, HD), jnp.float32),
    mesh=_mesh,
    scratch_types=[
        pltpu.VMEM((CH_A, K), jnp.int32),
        pltpu.VMEM((CH_A, K), jnp.int32),
        pltpu.VMEM((CH_A, K), jnp.int32),
        pltpu.VMEM((NBUF, K, HD), jnp.float32),
        pltpu.VMEM_SHARED((ACCR, HD), jnp.float32),
        pltpu.SemaphoreType.DMA,
        pltpu.SemaphoreType.DMA,
        pltpu.SemaphoreType.DMA,
        pltpu.SemaphoreType.DMA,
    ],
)
def _agg_kernel(y_hbm, src_hbm, dst_hbm, zeros_hbm, out_hbm,
                src_v, dst_raw, dst_v, bufs, acc, s0, s1, s2, s3):
    cid = lax.axis_index("c")
    sid = lax.axis_index("s")
    sems = (s0, s1, s2, s3)
    nsl = K // 16

    pltpu.sync_copy(src_hbm.at[sid], src_v)
    pltpu.sync_copy(dst_hbm.at[sid], dst_raw)

    fbase = cid * NPAD

    def _fix_src(t, _):
        j = t // nsl
        kk = (t % nsl) * 16
        src_v[j, pl.ds(kk, 16)] = src_v[j, pl.ds(kk, 16)] + fbase
        return _

    lax.fori_loop(0, CH_A * nsl, _fix_src, None)

    for q in range(NQUART):
        qbase = q * NQ

        def _fix_dst(t, _):
            j = t // nsl
            kk = (t % nsl) * 16
            loc = dst_raw[j, pl.ds(kk, 16)] - qbase
            ok = (loc >= 0) & (loc < NQ)
            dst_v[j, pl.ds(kk, 16)] = jnp.where(ok, loc, NQ)
            return _

        lax.fori_loop(0, CH_A * nsl, _fix_dst, None)

        pltpu.sync_copy(
            zeros_hbm.at[pl.ds(sid * SSTR, SSTR)],
            acc.at[pl.ds(sid * SSTR, SSTR)],
        )

        @pl.when(sid == 0)
        def _zero_dump():
            pltpu.sync_copy(zeros_hbm.at[pl.ds(0, 8)], acc.at[pl.ds(NQ, 8)])

        plsc.subcore_barrier()

        def _group(g, _):
            descs = []
            for b in range(NBUF):
                j = NBUF * g + b
                descs.append(
                    pltpu.async_copy(y_hbm.at[src_v.at[j]], bufs.at[b], sems[b])
                )
            for b in range(NBUF):
                descs[b].wait()
            for b in range(NBUF):
                j = NBUF * g + b
                pltpu.sync_copy(bufs.at[b], acc.at[dst_v.at[j]], add=True)
            return _

        lax.fori_loop(0, CH_A // NBUF, _group, None)
        plsc.subcore_barrier()
        pltpu.sync_copy(
            acc.at[pl.ds(sid * SSTR, SSTR)],
            out_hbm.at[pl.ds(cid * NPAD + qbase + sid * SSTR, SSTR)],
        )
        if q + 1 < NQUART:
            plsc.subcore_barrier()


def _row_mask(i, bm):
    rows = i * bm + lax.broadcasted_iota(jnp.int32, (bm, 1), 0)
    return rows < N


def _tc1_body(x_ref, w_ref, p0_ref, p1_ref, y_ref, dinv_ref):
    i = pl.program_id(0)
    xw = jnp.dot(x_ref[...], w_ref[...], preferred_element_type=jnp.float32)
    deg = 1.0 + p0_ref[:, 0:1] + p1_ref[:, 0:1]
    dinv = lax.rsqrt(deg)
    y = jnp.where(_row_mask(i, _BM), xw * dinv, 0.0)
    y_ref[0] = y[:, :HD]
    y_ref[1] = y[:, HD:]
    dinv_ref[...] = dinv


def _tc2_body(s_ref, y_ref, dinv_ref, b_ref, w_ref, o_ref):
    i = pl.program_id(0)
    s = jnp.concatenate([s_ref[0], s_ref[1]], axis=1)
    y = jnp.concatenate([y_ref[0], y_ref[1]], axis=1)
    h = jnp.maximum((s + y) * dinv_ref[...] + b_ref[...], 0.0)
    xw = jnp.dot(h, w_ref[...], preferred_element_type=jnp.float32)
    y2 = jnp.where(_row_mask(i, _BM), xw * dinv_ref[...], 0.0)
    o_ref[0] = y2[:, :HD]
    o_ref[1] = y2[:, HD:]


def _tc3_body(s_ref, y_ref, dinv_ref, b_ref, batch_ref, wfc_ref, bfc_ref, o_ref):
    s = jnp.concatenate([s_ref[0], s_ref[1]], axis=1)
    y = jnp.concatenate([y_ref[0], y_ref[1]], axis=1)
    h = jnp.maximum((s + y) * dinv_ref[...] + b_ref[...], 0.0)
    gids = lax.broadcasted_iota(jnp.int32, (1, NG), 1)
    m = (batch_ref[...] == gids).astype(jnp.float32)
    sums = lax.dot_general(
        m, h, (((0,), (0,)), ((), ())), preferred_element_type=jnp.float32
    )
    counts = lax.dot_general(
        m,
        jnp.ones((NPAD, 1), jnp.float32),
        (((0,), (0,)), ((), ())),
        preferred_element_type=jnp.float32,
    )
    g = sums / jnp.maximum(counts, 1.0)
    o_ref[...] = (
        jnp.dot(g, wfc_ref[...], preferred_element_type=jnp.float32) + bfc_ref[...]
    )


_BM = 640
_NBLK = NPAD // _BM


def _tc1(x, W1, degparts):
    return pl.pallas_call(
        _tc1_body,
        grid=(_NBLK,),
        in_specs=[
            pl.BlockSpec((_BM, D), lambda i: (i, 0)),
            pl.BlockSpec((D, D), lambda i: (0, 0)),
            pl.BlockSpec((_BM, LANES), lambda i: (i, 0)),
            pl.BlockSpec((_BM, LANES), lambda i: (i + _NBLK, 0)),
        ],
        out_specs=[
            pl.BlockSpec((NCORES, _BM, HD), lambda i: (0, i, 0)),
            pl.BlockSpec((_BM, 1), lambda i: (i, 0)),
        ],
        out_shape=[
            jax.ShapeDtypeStruct((NCORES, NPAD, HD), jnp.float32),
            jax.ShapeDtypeStruct((NPAD, 1), jnp.float32),
        ],
    )(x, W1, degparts, degparts)


def _tc2(S, y, dinv, b, W):
    return pl.pallas_call(
        _tc2_body,
        grid=(_NBLK,),
        in_specs=[
            pl.BlockSpec((NCORES, _BM, HD), lambda i: (0, i, 0)),
            pl.BlockSpec((NCORES, _BM, HD), lambda i: (0, i, 0)),
            pl.BlockSpec((_BM, 1), lambda i: (i, 0)),
            pl.BlockSpec((1, D), lambda i: (0, 0)),
            pl.BlockSpec((D, D), lambda i: (0, 0)),
        ],
        out_specs=pl.BlockSpec((NCORES, _BM, HD), lambda i: (0, i, 0)),
        out_shape=jax.ShapeDtypeStruct((NCORES, NPAD, HD), jnp.float32),
    )(S, y, dinv, b, W)


def _tc3(S, y, dinv, b, batch2, Wfc, bfc):
    return pl.pallas_call(
        _tc3_body,
        out_shape=jax.ShapeDtypeStruct((NG, NCLS), jnp.float32),
    )(S, y, dinv, b, batch2, Wfc, bfc)


def kernel(x, edge_index, batch, W1, b1, W2, b2, Wfc, bfc):
    src = edge_index[0].astype(jnp.int32)
    dst = edge_index[1].astype(jnp.int32)

    src_pad = jnp.concatenate(
        [src, jnp.full((EPAD - E,), SRC_PAD, jnp.int32)]
    ).reshape(NSUB, CH_A, K)
    dst_pad = jnp.concatenate(
        [dst, jnp.full((EPAD - E,), DST_PAD, jnp.int32)]
    ).reshape(NSUB, CH_A, K)

    x_pad = jnp.concatenate([x, jnp.zeros((NPAD - N, D), jnp.float32)], axis=0)
    batch_pad = jnp.concatenate(
        [batch.astype(jnp.int32), jnp.full((NPAD - N,), NG, jnp.int32)]
    ).reshape(NPAD, 1)
    zeros_h = jnp.zeros((NQ, HD), jnp.float32)
    pat = (
        jnp.arange(HD, dtype=jnp.int32)[None, :] // LANES
        == jnp.arange(8, dtype=jnp.int32)[:, None]
    ).astype(jnp.float32)

    degraw = _deg_kernel(pat, dst_pad, zeros_h)
    degparts = degraw.reshape(NCORES * NPAD, LANES)
    y1, dinv = _tc1(x_pad, W1, degparts)
    S1 = _agg_kernel(y1.reshape(NCORES * NPAD, HD), src_pad, dst_pad, zeros_h)
    y2 = _tc2(S1.reshape(NCORES, NPAD, HD), y1, dinv, b1.reshape(1, D), W2)
    S2 = _agg_kernel(y2.reshape(NCORES * NPAD, HD), src_pad, dst_pad, zeros_h)
    out = _tc3(
        S2.reshape(NCORES, NPAD, HD),
        y2,
        dinv,
        b2.reshape(1, D),
        batch_pad,
        Wfc,
        bfc.reshape(1, NCLS),
    )
    return out


# speedup vs baseline: 2.0114x; 1.3446x over previous
import functools

import jax
import jax.numpy as jnp
from jax import lax
from jax.experimental import pallas as pl
from jax.experimental.pallas import tpu as pltpu
from jax.experimental.pallas import tpu_sc as plsc

N = 10000
NPAD = 10240
E = 160000
D = 256
HD = 128
NG = 64
NCLS = 10
NCORES = 2
NSUB = 16
LANES = 16
NQUART = 4
NQ = NPAD // NQUART
ACCR = NQ + 8
SSTR = NQ // NSUB
K = 128
CH_A = 80
EPT = CH_A * K
EPAD = NSUB * EPT
NBUF = 4
GRP = NBUF * K
SEGTOT = EPT + NQUART * GRP
MAXCH = EPT // GRP * NBUF + NBUF * NQUART
SRC_PAD = N
DST_PAD = NPAD
PATR = 2048
DGR = NPAD // 8
DACCR = DGR + 8
DSTR = DGR // NSUB
CH_D = CH_A // NCORES

_mesh = plsc.VectorSubcoreMesh(core_axis_name="c", subcore_axis_name="s")


@functools.partial(
    pl.kernel,
    out_type=jax.ShapeDtypeStruct((NCORES * DGR, HD), jnp.float32),
    mesh=_mesh,
    scratch_types=[
        pltpu.VMEM((CH_A, K), jnp.int32),
        pltpu.VMEM((CH_D, K), jnp.int32),
        pltpu.VMEM((CH_D, K), jnp.int32),
        pltpu.VMEM((NBUF, K, HD), jnp.float32),
        pltpu.VMEM_SHARED((DACCR, HD), jnp.float32),
        pltpu.SemaphoreType.DMA,
        pltpu.SemaphoreType.DMA,
        pltpu.SemaphoreType.DMA,
        pltpu.SemaphoreType.DMA,
    ],
)
def _deg_kernel(pat_hbm, dst_hbm, zeros_hbm, out_hbm,
                dst_raw, gsel, gdst, bufs, acc, s0, s1, s2, s3):
    cid = lax.axis_index("c")
    sid = lax.axis_index("s")
    sems = (s0, s1, s2, s3)
    nsl = K // 16

    pltpu.sync_copy(dst_hbm.at[sid], dst_raw)

    def _fix(t, _):
        j = t // nsl
        kk = (t % nsl) * 16
        v = dst_raw[cid * CH_D + j, pl.ds(kk, 16)]
        gsel[j, pl.ds(kk, 16)] = jnp.bitwise_and(v, PATR - 1)
        gdst[j, pl.ds(kk, 16)] = lax.shift_right_logical(v, 3)
        return _

    lax.fori_loop(0, CH_D * nsl, _fix, None)

    pltpu.sync_copy(
        zeros_hbm.at[pl.ds(sid * DSTR, DSTR)],
        acc.at[pl.ds(sid * DSTR, DSTR)],
    )

    @pl.when(sid == 0)
    def _zero_pad():
        pltpu.sync_copy(zeros_hbm.at[pl.ds(0, 8)], acc.at[pl.ds(DGR, 8)])

    plsc.subcore_barrier()

    def _group(g, _):
        descs = []
        for b in range(NBUF):
            j = NBUF * g + b
            descs.append(
                pltpu.async_copy(pat_hbm.at[gsel.at[j]], bufs.at[b], sems[b])
            )
        for b in range(NBUF):
            descs[b].wait()
        for b in range(NBUF):
            j = NBUF * g + b
            pltpu.sync_copy(bufs.at[b], acc.at[gdst.at[j]], add=True)
        return _

    lax.fori_loop(0, CH_D // NBUF, _group, None)
    plsc.subcore_barrier()
    pltpu.sync_copy(
        acc.at[pl.ds(sid * DSTR, DSTR)],
        out_hbm.at[pl.ds(cid * DGR + sid * DSTR, DSTR)],
    )


@functools.partial(
    pl.kernel,
    out_type=jax.ShapeDtypeStruct((NCORES * NPAD, HD), jnp.float32),
    mesh=_mesh,
    scratch_types=[
        pltpu.VMEM((CH_A, K), jnp.int32),
        pltpu.VMEM((CH_A, K), jnp.int32),
        pltpu.VMEM((CH_A, K), jnp.int32),
        pltpu.VMEM((NBUF, K, HD), jnp.float32),
        pltpu.VMEM_SHARED((ACCR, HD), jnp.float32),
        pltpu.SemaphoreType.DMA,
        pltpu.SemaphoreType.DMA,
        pltpu.SemaphoreType.DMA,
        pltpu.SemaphoreType.DMA,
    ],
)
def _agg_kernel(y_hbm, src_hbm, dst_hbm, zeros_hbm, out_hbm,
                src_v, dst_raw, dst_v, bufs, acc, s0, s1, s2, s3):
    cid = lax.axis_index("c")
    sid = lax.axis_index("s")
    sems = (s0, s1, s2, s3)
    nsl = K // 16

    pltpu.sync_copy(src_hbm.at[sid], src_v)
    pltpu.sync_copy(dst_hbm.at[sid], dst_raw)

    def _add_base(t, _):
        j = t // nsl
        kk = (t % nsl) * 16
        src_v[j, pl.ds(kk, 16)] = src_v[j, pl.ds(kk, 16)] + NPAD
        return _

    @pl.when(cid == 1)
    def _shift_src():
        lax.fori_loop(0, CH_A * nsl, _add_base, None)

    for q in range(NQUART):
        qbase = q * NQ

        def _fix_dst(t, _):
            j = t // nsl
            kk = (t % nsl) * 16
            loc = dst_raw[j, pl.ds(kk, 16)] - qbase
            ok = (loc >= 0) & (loc < NQ)
            dst_v[j, pl.ds(kk, 16)] = jnp.where(ok, loc, NQ)
            return _

        lax.fori_loop(0, CH_A * nsl, _fix_dst, None)

        pltpu.sync_copy(
            zeros_hbm.at[pl.ds(sid * SSTR, SSTR)],
            acc.at[pl.ds(sid * SSTR, SSTR)],
        )

        @pl.when(sid == 0)
        def _zero_dump():
            pltpu.sync_copy(zeros_hbm.at[pl.ds(0, 8)], acc.at[pl.ds(NQ, 8)])

        plsc.subcore_barrier()

        def _group(g, _):
            descs = []
            for b in range(NBUF):
                j = NBUF * g + b
                descs.append(
                    pltpu.async_copy(y_hbm.at[src_v.at[j]], bufs.at[b], sems[b])
                )
            for b in range(NBUF):
                descs[b].wait()
            for b in range(NBUF):
                j = NBUF * g + b
                pltpu.sync_copy(bufs.at[b], acc.at[dst_v.at[j]], add=True)
            return _

        lax.fori_loop(0, CH_A // NBUF, _group, None)
        plsc.subcore_barrier()
        pltpu.sync_copy(
            acc.at[pl.ds(sid * SSTR, SSTR)],
            out_hbm.at[pl.ds(cid * NPAD + qbase + sid * SSTR, SSTR)],
        )
        if q + 1 < NQUART:
            plsc.subcore_barrier()


def _row_mask(i, bm):
    rows = i * bm + lax.broadcasted_iota(jnp.int32, (bm, 1), 0)
    return rows < N


def _tc1_body(x_ref, w_ref, p0_ref, p1_ref, y_ref, dinv_ref):
    i = pl.program_id(0)
    xw = jnp.dot(x_ref[...], w_ref[...], preferred_element_type=jnp.float32)
    deg = 1.0 + p0_ref[:, 0:1] + p1_ref[:, 0:1]
    dinv = lax.rsqrt(deg)
    y = jnp.where(_row_mask(i, _BM), xw * dinv, 0.0)
    y_ref[0] = y[:, :HD]
    y_ref[1] = y[:, HD:]
    dinv_ref[...] = dinv


def _tc2_body(s_ref, y_ref, dinv_ref, b_ref, w_ref, o_ref):
    i = pl.program_id(0)
    s = jnp.concatenate([s_ref[0], s_ref[1]], axis=1)
    y = jnp.concatenate([y_ref[0], y_ref[1]], axis=1)
    h = jnp.maximum((s + y) * dinv_ref[...] + b_ref[...], 0.0)
    xw = jnp.dot(h, w_ref[...], preferred_element_type=jnp.float32)
    y2 = jnp.where(_row_mask(i, _BM), xw * dinv_ref[...], 0.0)
    o_ref[0] = y2[:, :HD]
    o_ref[1] = y2[:, HD:]


def _tc3_body(s_ref, y_ref, dinv_ref, b_ref, batch_ref, wfc_ref, bfc_ref, o_ref):
    s = jnp.concatenate([s_ref[0], s_ref[1]], axis=1)
    y = jnp.concatenate([y_ref[0], y_ref[1]], axis=1)
    h = jnp.maximum((s + y) * dinv_ref[...] + b_ref[...], 0.0)
    gids = lax.broadcasted_iota(jnp.int32, (1, NG), 1)
    m = (batch_ref[...] == gids).astype(jnp.float32)
    sums = lax.dot_general(
        m, h, (((0,), (0,)), ((), ())), preferred_element_type=jnp.float32
    )
    counts = lax.dot_general(
        m,
        jnp.ones((NPAD, 1), jnp.float32),
        (((0,), (0,)), ((), ())),
        preferred_element_type=jnp.float32,
    )
    g = sums / jnp.maximum(counts, 1.0)
    o_ref[...] = (
        jnp.dot(g, wfc_ref[...], preferred_element_type=jnp.float32) + bfc_ref[...]
    )


_BM = 640
_NBLK = NPAD // _BM


def _tc1(x, W1, degparts):
    return pl.pallas_call(
        _tc1_body,
        grid=(_NBLK,),
        in_specs=[
            pl.BlockSpec((_BM, D), lambda i: (i, 0)),
            pl.BlockSpec((D, D), lambda i: (0, 0)),
            pl.BlockSpec((_BM, LANES), lambda i: (i, 0)),
            pl.BlockSpec((_BM, LANES), lambda i: (i + _NBLK, 0)),
        ],
        out_specs=[
            pl.BlockSpec((NCORES, _BM, HD), lambda i: (0, i, 0)),
            pl.BlockSpec((_BM, 1), lambda i: (i, 0)),
        ],
        out_shape=[
            jax.ShapeDtypeStruct((NCORES, NPAD, HD), jnp.float32),
            jax.ShapeDtypeStruct((NPAD, 1), jnp.float32),
        ],
    )(x, W1, degparts, degparts)


def _tc2(S, y, dinv, b, W):
    return pl.pallas_call(
        _tc2_body,
        grid=(_NBLK,),
        in_specs=[
            pl.BlockSpec((NCORES, _BM, HD), lambda i: (0, i, 0)),
            pl.BlockSpec((NCORES, _BM, HD), lambda i: (0, i, 0)),
            pl.BlockSpec((_BM, 1), lambda i: (i, 0)),
            pl.BlockSpec((1, D), lambda i: (0, 0)),
            pl.BlockSpec((D, D), lambda i: (0, 0)),
        ],
        out_specs=pl.BlockSpec((NCORES, _BM, HD), lambda i: (0, i, 0)),
        out_shape=jax.ShapeDtypeStruct((NCORES, NPAD, HD), jnp.float32),
    )(S, y, dinv, b, W)


def _tc3(S, y, dinv, b, batch2, Wfc, bfc):
    return pl.pallas_call(
        _tc3_body,
        out_shape=jax.ShapeDtypeStruct((NG, NCLS), jnp.float32),
    )(S, y, dinv, b, batch2, Wfc, bfc)


def kernel(x, edge_index, batch, W1, b1, W2, b2, Wfc, bfc):
    src = edge_index[0].astype(jnp.int32)
    dst = edge_index[1].astype(jnp.int32)

    src_pad = jnp.concatenate(
        [src, jnp.full((EPAD - E,), SRC_PAD, jnp.int32)]
    ).reshape(NSUB, CH_A, K)
    dst_pad = jnp.concatenate(
        [dst, jnp.full((EPAD - E,), DST_PAD, jnp.int32)]
    ).reshape(NSUB, CH_A, K)

    x_pad = jnp.concatenate([x, jnp.zeros((NPAD - N, D), jnp.float32)], axis=0)
    batch_pad = jnp.concatenate(
        [batch.astype(jnp.int32), jnp.full((NPAD - N,), NG, jnp.int32)]
    ).reshape(NPAD, 1)
    zeros_h = jnp.zeros((NQ, HD), jnp.float32)
    pat = (
        jnp.arange(HD, dtype=jnp.int32)[None, :] // LANES
        == (jnp.arange(PATR, dtype=jnp.int32) % 8)[:, None]
    ).astype(jnp.float32)
    degraw = _deg_kernel(pat, dst_pad, zeros_h)
    degparts = degraw.reshape(NCORES * NPAD, LANES)
    y1, dinv = _tc1(x_pad, W1, degparts)
    S1 = _agg_kernel(y1.reshape(NCORES * NPAD, HD), src_pad, dst_pad, zeros_h)
    y2 = _tc2(S1.reshape(NCORES, NPAD, HD), y1, dinv, b1.reshape(1, D), W2)
    S2 = _agg_kernel(y2.reshape(NCORES * NPAD, HD), src_pad, dst_pad, zeros_h)
    out = _tc3(
        S2.reshape(NCORES, NPAD, HD),
        y2,
        dinv,
        b2.reshape(1, D),
        batch_pad,
        Wfc,
        bfc.reshape(1, NCLS),
    )
    return out


# speedup vs baseline: 2.2138x; 1.1006x over previous
import functools

import jax
import jax.numpy as jnp
from jax import lax
from jax.experimental import pallas as pl
from jax.experimental.pallas import tpu as pltpu
from jax.experimental.pallas import tpu_sc as plsc

N = 10000
NPAD = 10240
E = 160000
D = 256
HD = 128
NG = 64
NCLS = 10
NCORES = 2
NSUB = 16
LANES = 16
NQUART = 4
NQ = NPAD // NQUART
ACCR = NQ + 8
SSTR = NQ // NSUB
K = 128
CH_A = 80
EPT = CH_A * K
EPAD = NSUB * EPT
NBUF = 4
GRP = NBUF * K
SEGTOT = EPT + NQUART * GRP
MAXCH = EPT // GRP * NBUF + NBUF * NQUART
SRC_PAD = N
DST_PAD = NPAD
PATR = 2048
DGR = NPAD // 8
DACCR = DGR + 8
DSTR = DGR // NSUB
CH_D = CH_A // NCORES

_mesh = plsc.VectorSubcoreMesh(core_axis_name="c", subcore_axis_name="s")


@functools.partial(
    pl.kernel,
    out_type=jax.ShapeDtypeStruct((NCORES * DGR, HD), jnp.float32),
    mesh=_mesh,
    scratch_types=[
        pltpu.VMEM((CH_A, K), jnp.int32),
        pltpu.VMEM((CH_D, K), jnp.int32),
        pltpu.VMEM((CH_D, K), jnp.int32),
        pltpu.VMEM((NBUF, K, HD), jnp.float32),
        pltpu.VMEM_SHARED((DACCR, HD), jnp.float32),
        pltpu.SemaphoreType.DMA,
        pltpu.SemaphoreType.DMA,
        pltpu.SemaphoreType.DMA,
        pltpu.SemaphoreType.DMA,
    ],
)
def _deg_kernel(pat_hbm, dst_hbm, zeros_hbm, out_hbm,
                dst_raw, gsel, gdst, bufs, acc, s0, s1, s2, s3):
    cid = lax.axis_index("c")
    sid = lax.axis_index("s")
    sems = (s0, s1, s2, s3)
    nsl = K // 16

    pltpu.sync_copy(dst_hbm.at[sid], dst_raw)

    def _fix(t, _):
        j = t // nsl
        kk = (t % nsl) * 16
        v = dst_raw[cid * CH_D + j, pl.ds(kk, 16)]
        gsel[j, pl.ds(kk, 16)] = jnp.bitwise_and(v, PATR - 1)
        gdst[j, pl.ds(kk, 16)] = lax.shift_right_logical(v, 3)
        return _

    lax.fori_loop(0, CH_D * nsl, _fix, None)

    pltpu.sync_copy(
        zeros_hbm.at[pl.ds(sid * DSTR, DSTR)],
        acc.at[pl.ds(sid * DSTR, DSTR)],
    )

    @pl.when(sid == 0)
    def _zero_pad():
        pltpu.sync_copy(zeros_hbm.at[pl.ds(0, 8)], acc.at[pl.ds(DGR, 8)])

    plsc.subcore_barrier()

    def _group(g, _):
        descs = []
        for b in range(NBUF):
            j = NBUF * g + b
            descs.append(
                pltpu.async_copy(pat_hbm.at[gsel.at[j]], bufs.at[b], sems[b])
            )
        for b in range(NBUF):
            descs[b].wait()
        for b in range(NBUF):
            j = NBUF * g + b
            pltpu.sync_copy(bufs.at[b], acc.at[gdst.at[j]], add=True)
        return _

    lax.fori_loop(0, CH_D // NBUF, _group, None)
    plsc.subcore_barrier()
    pltpu.sync_copy(
        acc.at[pl.ds(sid * DSTR, DSTR)],
        out_hbm.at[pl.ds(cid * DGR + sid * DSTR, DSTR)],
    )


@functools.partial(
    pl.kernel,
    out_type=jax.ShapeDtypeStruct((NCORES * NPAD, HD), jnp.float32),
    mesh=_mesh,
    scratch_types=[
        pltpu.VMEM((CH_A, K), jnp.int32),
        pltpu.VMEM((CH_A, K), jnp.int32),
        pltpu.VMEM((CH_A, K), jnp.int32),
        pltpu.VMEM((NBUF, K, HD), jnp.float32),
        pltpu.VMEM_SHARED((ACCR, HD), jnp.float32),
        pltpu.SemaphoreType.DMA,
        pltpu.SemaphoreType.DMA,
        pltpu.SemaphoreType.DMA,
        pltpu.SemaphoreType.DMA,
    ],
)
def _agg_kernel(y_hbm, src_hbm, dst_hbm, zeros_hbm, out_hbm,
                src_v, dst_raw, dst_v, bufs, acc, s0, s1, s2, s3):
    cid = lax.axis_index("c")
    sid = lax.axis_index("s")
    sems = (s0, s1, s2, s3)
    nsl = K // 16

    pltpu.sync_copy(src_hbm.at[sid], src_v)
    pltpu.sync_copy(dst_hbm.at[sid], dst_raw)

    def _add_base(t, _):
        j = t // nsl
        kk = (t % nsl) * 16
        src_v[j, pl.ds(kk, 16)] = src_v[j, pl.ds(kk, 16)] + NPAD
        return _

    @pl.when(cid == 1)
    def _shift_src():
        lax.fori_loop(0, CH_A * nsl, _add_base, None)

    for q in range(NQUART):
        qbase = q * NQ

        dump_vec = NQ + jnp.bitwise_and(
            lax.broadcasted_iota(jnp.int32, (16,), 0), 7
        )

        def _fix_dst(t, _):
            j = t // nsl
            kk = (t % nsl) * 16
            loc = dst_raw[j, pl.ds(kk, 16)] - qbase
            ok = (loc >= 0) & (loc < NQ)
            dst_v[j, pl.ds(kk, 16)] = jnp.where(ok, loc, dump_vec)
            return _

        lax.fori_loop(0, CH_A * nsl, _fix_dst, None)

        pltpu.sync_copy(
            zeros_hbm.at[pl.ds(sid * SSTR, SSTR)],
            acc.at[pl.ds(sid * SSTR, SSTR)],
        )

        @pl.when(sid == 0)
        def _zero_dump():
            pltpu.sync_copy(zeros_hbm.at[pl.ds(0, 8)], acc.at[pl.ds(NQ, 8)])

        plsc.subcore_barrier()

        def _group(g, _):
            descs = []
            for b in range(NBUF):
                j = NBUF * g + b
                descs.append(
                    pltpu.async_copy(y_hbm.at[src_v.at[j]], bufs.at[b], sems[b])
                )
            for b in range(NBUF):
                descs[b].wait()
            for b in range(NBUF):
                j = NBUF * g + b
                pltpu.sync_copy(bufs.at[b], acc.at[dst_v.at[j]], add=True)
            return _

        lax.fori_loop(0, CH_A // NBUF, _group, None)
        plsc.subcore_barrier()
        pltpu.sync_copy(
            acc.at[pl.ds(sid * SSTR, SSTR)],
            out_hbm.at[pl.ds(cid * NPAD + qbase + sid * SSTR, SSTR)],
        )
        if q + 1 < NQUART:
            plsc.subcore_barrier()


def _row_mask(i, bm):
    rows = i * bm + lax.broadcasted_iota(jnp.int32, (bm, 1), 0)
    return rows < N


def _tc1_body(x_ref, w_ref, p0_ref, p1_ref, y_ref, dinv_ref):
    i = pl.program_id(0)
    xw = jnp.dot(x_ref[...], w_ref[...], preferred_element_type=jnp.float32)
    deg = 1.0 + p0_ref[:, 0:1] + p1_ref[:, 0:1]
    dinv = lax.rsqrt(deg)
    y = jnp.where(_row_mask(i, _BM), xw * dinv, 0.0)
    y_ref[0] = y[:, :HD]
    y_ref[1] = y[:, HD:]
    dinv_ref[...] = dinv


def _tc2_body(s_ref, y_ref, dinv_ref, b_ref, w_ref, o_ref):
    i = pl.program_id(0)
    s = jnp.concatenate([s_ref[0], s_ref[1]], axis=1)
    y = jnp.concatenate([y_ref[0], y_ref[1]], axis=1)
    h = jnp.maximum((s + y) * dinv_ref[...] + b_ref[...], 0.0)
    xw = jnp.dot(h, w_ref[...], preferred_element_type=jnp.float32)
    y2 = jnp.where(_row_mask(i, _BM), xw * dinv_ref[...], 0.0)
    o_ref[0] = y2[:, :HD]
    o_ref[1] = y2[:, HD:]


def _tc3_body(s_ref, y_ref, dinv_ref, b_ref, batch_ref, wfc_ref, bfc_ref, o_ref):
    s = jnp.concatenate([s_ref[0], s_ref[1]], axis=1)
    y = jnp.concatenate([y_ref[0], y_ref[1]], axis=1)
    h = jnp.maximum((s + y) * dinv_ref[...] + b_ref[...], 0.0)
    gids = lax.broadcasted_iota(jnp.int32, (1, NG), 1)
    m = (batch_ref[...] == gids).astype(jnp.float32)
    sums = lax.dot_general(
        m, h, (((0,), (0,)), ((), ())), preferred_element_type=jnp.float32
    )
    counts = lax.dot_general(
        m,
        jnp.ones((NPAD, 1), jnp.float32),
        (((0,), (0,)), ((), ())),
        preferred_element_type=jnp.float32,
    )
    g = sums / jnp.maximum(counts, 1.0)
    o_ref[...] = (
        jnp.dot(g, wfc_ref[...], preferred_element_type=jnp.float32) + bfc_ref[...]
    )


_BM = 640
_NBLK = NPAD // _BM


def _tc1(x, W1, degparts):
    return pl.pallas_call(
        _tc1_body,
        grid=(_NBLK,),
        in_specs=[
            pl.BlockSpec((_BM, D), lambda i: (i, 0)),
            pl.BlockSpec((D, D), lambda i: (0, 0)),
            pl.BlockSpec((_BM, LANES), lambda i: (i, 0)),
            pl.BlockSpec((_BM, LANES), lambda i: (i + _NBLK, 0)),
        ],
        out_specs=[
            pl.BlockSpec((NCORES, _BM, HD), lambda i: (0, i, 0)),
            pl.BlockSpec((_BM, 1), lambda i: (i, 0)),
        ],
        out_shape=[
            jax.ShapeDtypeStruct((NCORES, NPAD, HD), jnp.float32),
            jax.ShapeDtypeStruct((NPAD, 1), jnp.float32),
        ],
    )(x, W1, degparts, degparts)


def _tc2(S, y, dinv, b, W):
    return pl.pallas_call(
        _tc2_body,
        grid=(_NBLK,),
        in_specs=[
            pl.BlockSpec((NCORES, _BM, HD), lambda i: (0, i, 0)),
            pl.BlockSpec((NCORES, _BM, HD), lambda i: (0, i, 0)),
            pl.BlockSpec((_BM, 1), lambda i: (i, 0)),
            pl.BlockSpec((1, D), lambda i: (0, 0)),
            pl.BlockSpec((D, D), lambda i: (0, 0)),
        ],
        out_specs=pl.BlockSpec((NCORES, _BM, HD), lambda i: (0, i, 0)),
        out_shape=jax.ShapeDtypeStruct((NCORES, NPAD, HD), jnp.float32),
    )(S, y, dinv, b, W)


def _tc3(S, y, dinv, b, batch2, Wfc, bfc):
    return pl.pallas_call(
        _tc3_body,
        out_shape=jax.ShapeDtypeStruct((NG, NCLS), jnp.float32),
    )(S, y, dinv, b, batch2, Wfc, bfc)


def kernel(x, edge_index, batch, W1, b1, W2, b2, Wfc, bfc):
    src = edge_index[0].astype(jnp.int32)
    dst = edge_index[1].astype(jnp.int32)

    src_pad = jnp.concatenate(
        [src, jnp.full((EPAD - E,), SRC_PAD, jnp.int32)]
    ).reshape(NSUB, CH_A, K)
    dst_pad = jnp.concatenate(
        [dst, jnp.full((EPAD - E,), DST_PAD, jnp.int32)]
    ).reshape(NSUB, CH_A, K)

    x_pad = jnp.concatenate([x, jnp.zeros((NPAD - N, D), jnp.float32)], axis=0)
    batch_pad = jnp.concatenate(
        [batch.astype(jnp.int32), jnp.full((NPAD - N,), NG, jnp.int32)]
    ).reshape(NPAD, 1)
    zeros_h = jnp.zeros((NQ, HD), jnp.float32)
    pat = (
        jnp.arange(HD, dtype=jnp.int32)[None, :] // LANES
        == (jnp.arange(PATR, dtype=jnp.int32) % 8)[:, None]
    ).astype(jnp.float32)
    degraw = _deg_kernel(pat, dst_pad, zeros_h)
    degparts = degraw.reshape(NCORES * NPAD, LANES)
    y1, dinv = _tc1(x_pad, W1, degparts)
    S1 = _agg_kernel(y1.reshape(NCORES * NPAD, HD), src_pad, dst_pad, zeros_h)
    y2 = _tc2(S1.reshape(NCORES, NPAD, HD), y1, dinv, b1.reshape(1, D), W2)
    S2 = _agg_kernel(y2.reshape(NCORES * NPAD, HD), src_pad, dst_pad, zeros_h)
    out = _tc3(
        S2.reshape(NCORES, NPAD, HD),
        y2,
        dinv,
        b2.reshape(1, D),
        batch_pad,
        Wfc,
        bfc.reshape(1, NCLS),
    )
    return out


# speedup vs baseline: 2.2149x; 1.0005x over previous
import functools

import jax
import jax.numpy as jnp
from jax import lax
from jax.experimental import pallas as pl
from jax.experimental.pallas import tpu as pltpu
from jax.experimental.pallas import tpu_sc as plsc

N = 10000
NPAD = 10240
E = 160000
D = 256
HD = 128
NG = 64
NCLS = 10
NCORES = 2
NSUB = 16
LANES = 16
NQUART = 4
NQ = NPAD // NQUART
ACCR = NQ + 16
SSTR = NQ // NSUB
K = 128
CH_A = 80
EPT = CH_A * K
EPAD = NSUB * EPT
NBUF = 4
GRP = NBUF * K
SEGTOT = EPT + NQUART * GRP
MAXCH = EPT // GRP * NBUF + NBUF * NQUART
SRC_PAD = N
DST_PAD = NPAD
PATR = 2048
DGR = NPAD // 8
DACCR = DGR + 8
DSTR = DGR // NSUB
CH_D = CH_A // NCORES

_mesh = plsc.VectorSubcoreMesh(core_axis_name="c", subcore_axis_name="s")


@functools.partial(
    pl.kernel,
    out_type=jax.ShapeDtypeStruct((NCORES * DGR, HD), jnp.float32),
    mesh=_mesh,
    scratch_types=[
        pltpu.VMEM((CH_A, K), jnp.int32),
        pltpu.VMEM((CH_D, K), jnp.int32),
        pltpu.VMEM((CH_D, K), jnp.int32),
        pltpu.VMEM((NBUF, K, HD), jnp.float32),
        pltpu.VMEM_SHARED((DACCR, HD), jnp.float32),
        pltpu.SemaphoreType.DMA,
        pltpu.SemaphoreType.DMA,
        pltpu.SemaphoreType.DMA,
        pltpu.SemaphoreType.DMA,
    ],
)
def _deg_kernel(pat_hbm, dst_hbm, zeros_hbm, out_hbm,
                dst_raw, gsel, gdst, bufs, acc, s0, s1, s2, s3):
    cid = lax.axis_index("c")
    sid = lax.axis_index("s")
    sems = (s0, s1, s2, s3)
    nsl = K // 16

    pltpu.sync_copy(dst_hbm.at[sid], dst_raw)

    def _fix(t, _):
        j = t // nsl
        kk = (t % nsl) * 16
        v = dst_raw[cid * CH_D + j, pl.ds(kk, 16)]
        gsel[j, pl.ds(kk, 16)] = jnp.bitwise_and(v, PATR - 1)
        gdst[j, pl.ds(kk, 16)] = lax.shift_right_logical(v, 3)
        return _

    lax.fori_loop(0, CH_D * nsl, _fix, None)

    pltpu.sync_copy(
        zeros_hbm.at[pl.ds(sid * DSTR, DSTR)],
        acc.at[pl.ds(sid * DSTR, DSTR)],
    )

    @pl.when(sid == 0)
    def _zero_pad():
        pltpu.sync_copy(zeros_hbm.at[pl.ds(0, 8)], acc.at[pl.ds(DGR, 8)])

    plsc.subcore_barrier()

    def _group(g, _):
        descs = []
        for b in range(NBUF):
            j = NBUF * g + b
            descs.append(
                pltpu.async_copy(pat_hbm.at[gsel.at[j]], bufs.at[b], sems[b])
            )
        for b in range(NBUF):
            descs[b].wait()
        for b in range(NBUF):
            j = NBUF * g + b
            pltpu.sync_copy(bufs.at[b], acc.at[gdst.at[j]], add=True)
        return _

    lax.fori_loop(0, CH_D // NBUF, _group, None)
    plsc.subcore_barrier()
    pltpu.sync_copy(
        acc.at[pl.ds(sid * DSTR, DSTR)],
        out_hbm.at[pl.ds(cid * DGR + sid * DSTR, DSTR)],
    )


@functools.partial(
    pl.kernel,
    out_type=jax.ShapeDtypeStruct((NCORES * NPAD, HD), jnp.float32),
    mesh=_mesh,
    scratch_types=[
        pltpu.VMEM((CH_A, K), jnp.int32),
        pltpu.VMEM((CH_A, K), jnp.int32),
        pltpu.VMEM((CH_A, K), jnp.int32),
        pltpu.VMEM((NBUF, K, HD), jnp.float32),
        pltpu.VMEM_SHARED((ACCR, HD), jnp.float32),
        pltpu.SemaphoreType.DMA,
        pltpu.SemaphoreType.DMA,
        pltpu.SemaphoreType.DMA,
        pltpu.SemaphoreType.DMA,
    ],
)
def _agg_kernel(y_hbm, src_hbm, dst_hbm, zeros_hbm, out_hbm,
                src_v, dst_raw, dst_v, bufs, acc, s0, s1, s2, s3):
    cid = lax.axis_index("c")
    sid = lax.axis_index("s")
    sems = (s0, s1, s2, s3)
    nsl = K // 16

    pltpu.sync_copy(src_hbm.at[sid], src_v)
    pltpu.sync_copy(dst_hbm.at[sid], dst_raw)

    def _add_base(t, _):
        j = t // nsl
        kk = (t % nsl) * 16
        src_v[j, pl.ds(kk, 16)] = src_v[j, pl.ds(kk, 16)] + NPAD
        return _

    @pl.when(cid == 1)
    def _shift_src():
        lax.fori_loop(0, CH_A * nsl, _add_base, None)

    for q in range(NQUART):
        qbase = q * NQ

        dump_vec = NQ + lax.broadcasted_iota(
            jnp.int32, (16,), 0
        )

        def _fix_dst(t, _):
            j = t // nsl
            kk = (t % nsl) * 16
            loc = dst_raw[j, pl.ds(kk, 16)] - qbase
            ok = (loc >= 0) & (loc < NQ)
            dst_v[j, pl.ds(kk, 16)] = jnp.where(ok, loc, dump_vec)
            return _

        lax.fori_loop(0, CH_A * nsl, _fix_dst, None)

        pltpu.sync_copy(
            zeros_hbm.at[pl.ds(sid * SSTR, SSTR)],
            acc.at[pl.ds(sid * SSTR, SSTR)],
        )

        @pl.when(sid == 0)
        def _zero_dump():
            pltpu.sync_copy(zeros_hbm.at[pl.ds(0, 16)], acc.at[pl.ds(NQ, 16)])

        plsc.subcore_barrier()

        def _group(g, _):
            descs = []
            for b in range(NBUF):
                j = NBUF * g + b
                descs.append(
                    pltpu.async_copy(y_hbm.at[src_v.at[j]], bufs.at[b], sems[b])
                )
            for b in range(NBUF):
                descs[b].wait()
            for b in range(NBUF):
                j = NBUF * g + b
                pltpu.sync_copy(bufs.at[b], acc.at[dst_v.at[j]], add=True)
            return _

        lax.fori_loop(0, CH_A // NBUF, _group, None)
        plsc.subcore_barrier()
        pltpu.sync_copy(
            acc.at[pl.ds(sid * SSTR, SSTR)],
            out_hbm.at[pl.ds(cid * NPAD + qbase + sid * SSTR, SSTR)],
        )
        if q + 1 < NQUART:
            plsc.subcore_barrier()


def _row_mask(i, bm):
    rows = i * bm + lax.broadcasted_iota(jnp.int32, (bm, 1), 0)
    return rows < N


def _tc1_body(x_ref, w_ref, p0_ref, p1_ref, y_ref, dinv_ref):
    i = pl.program_id(0)
    xw = jnp.dot(x_ref[...], w_ref[...], preferred_element_type=jnp.float32)
    deg = 1.0 + p0_ref[:, 0:1] + p1_ref[:, 0:1]
    dinv = lax.rsqrt(deg)
    y = jnp.where(_row_mask(i, _BM), xw * dinv, 0.0)
    y_ref[0] = y[:, :HD]
    y_ref[1] = y[:, HD:]
    dinv_ref[...] = dinv


def _tc2_body(s_ref, y_ref, dinv_ref, b_ref, w_ref, o_ref):
    i = pl.program_id(0)
    s = jnp.concatenate([s_ref[0], s_ref[1]], axis=1)
    y = jnp.concatenate([y_ref[0], y_ref[1]], axis=1)
    h = jnp.maximum((s + y) * dinv_ref[...] + b_ref[...], 0.0)
    xw = jnp.dot(h, w_ref[...], preferred_element_type=jnp.float32)
    y2 = jnp.where(_row_mask(i, _BM), xw * dinv_ref[...], 0.0)
    o_ref[0] = y2[:, :HD]
    o_ref[1] = y2[:, HD:]


def _tc3_body(s_ref, y_ref, dinv_ref, b_ref, batch_ref, wfc_ref, bfc_ref, o_ref):
    s = jnp.concatenate([s_ref[0], s_ref[1]], axis=1)
    y = jnp.concatenate([y_ref[0], y_ref[1]], axis=1)
    h = jnp.maximum((s + y) * dinv_ref[...] + b_ref[...], 0.0)
    gids = lax.broadcasted_iota(jnp.int32, (1, NG), 1)
    m = (batch_ref[...] == gids).astype(jnp.float32)
    sums = lax.dot_general(
        m, h, (((0,), (0,)), ((), ())), preferred_element_type=jnp.float32
    )
    counts = lax.dot_general(
        m,
        jnp.ones((NPAD, 1), jnp.float32),
        (((0,), (0,)), ((), ())),
        preferred_element_type=jnp.float32,
    )
    g = sums / jnp.maximum(counts, 1.0)
    o_ref[...] = (
        jnp.dot(g, wfc_ref[...], preferred_element_type=jnp.float32) + bfc_ref[...]
    )


_BM = 640
_NBLK = NPAD // _BM


def _tc1(x, W1, degparts):
    return pl.pallas_call(
        _tc1_body,
        grid=(_NBLK,),
        in_specs=[
            pl.BlockSpec((_BM, D), lambda i: (i, 0)),
            pl.BlockSpec((D, D), lambda i: (0, 0)),
            pl.BlockSpec((_BM, LANES), lambda i: (i, 0)),
            pl.BlockSpec((_BM, LANES), lambda i: (i + _NBLK, 0)),
        ],
        out_specs=[
            pl.BlockSpec((NCORES, _BM, HD), lambda i: (0, i, 0)),
            pl.BlockSpec((_BM, 1), lambda i: (i, 0)),
        ],
        out_shape=[
            jax.ShapeDtypeStruct((NCORES, NPAD, HD), jnp.float32),
            jax.ShapeDtypeStruct((NPAD, 1), jnp.float32),
        ],
    )(x, W1, degparts, degparts)


def _tc2(S, y, dinv, b, W):
    return pl.pallas_call(
        _tc2_body,
        grid=(_NBLK,),
        in_specs=[
            pl.BlockSpec((NCORES, _BM, HD), lambda i: (0, i, 0)),
            pl.BlockSpec((NCORES, _BM, HD), lambda i: (0, i, 0)),
            pl.BlockSpec((_BM, 1), lambda i: (i, 0)),
            pl.BlockSpec((1, D), lambda i: (0, 0)),
            pl.BlockSpec((D, D), lambda i: (0, 0)),
        ],
        out_specs=pl.BlockSpec((NCORES, _BM, HD), lambda i: (0, i, 0)),
        out_shape=jax.ShapeDtypeStruct((NCORES, NPAD, HD), jnp.float32),
    )(S, y, dinv, b, W)


def _tc3(S, y, dinv, b, batch2, Wfc, bfc):
    return pl.pallas_call(
        _tc3_body,
        out_shape=jax.ShapeDtypeStruct((NG, NCLS), jnp.float32),
    )(S, y, dinv, b, batch2, Wfc, bfc)


def kernel(x, edge_index, batch, W1, b1, W2, b2, Wfc, bfc):
    src = edge_index[0].astype(jnp.int32)
    dst = edge_index[1].astype(jnp.int32)

    src_pad = jnp.concatenate(
        [src, jnp.full((EPAD - E,), SRC_PAD, jnp.int32)]
    ).reshape(NSUB, CH_A, K)
    dst_pad = jnp.concatenate(
        [dst, jnp.full((EPAD - E,), DST_PAD, jnp.int32)]
    ).reshape(NSUB, CH_A, K)

    x_pad = jnp.concatenate([x, jnp.zeros((NPAD - N, D), jnp.float32)], axis=0)
    batch_pad = jnp.concatenate(
        [batch.astype(jnp.int32), jnp.full((NPAD - N,), NG, jnp.int32)]
    ).reshape(NPAD, 1)
    zeros_h = jnp.zeros((NQ, HD), jnp.float32)
    pat = (
        jnp.arange(HD, dtype=jnp.int32)[None, :] // LANES
        == (jnp.arange(PATR, dtype=jnp.int32) % 8)[:, None]
    ).astype(jnp.float32)
    degraw = _deg_kernel(pat, dst_pad, zeros_h)
    degparts = degraw.reshape(NCORES * NPAD, LANES)
    y1, dinv = _tc1(x_pad, W1, degparts)
    S1 = _agg_kernel(y1.reshape(NCORES * NPAD, HD), src_pad, dst_pad, zeros_h)
    y2 = _tc2(S1.reshape(NCORES, NPAD, HD), y1, dinv, b1.reshape(1, D), W2)
    S2 = _agg_kernel(y2.reshape(NCORES * NPAD, HD), src_pad, dst_pad, zeros_h)
    out = _tc3(
        S2.reshape(NCORES, NPAD, HD),
        y2,
        dinv,
        b2.reshape(1, D),
        batch_pad,
        Wfc,
        bfc.reshape(1, NCLS),
    )
    return out
